# Initial kernel scaffold; baseline (speedup 1.0000x reference)
import jax
import jax.numpy as jnp
from jax.experimental import pallas as pl


def kernel(queries, keys, values):
    raise NotImplementedError("write your pallas kernel here")


# speedup vs baseline: 3.6396x; 3.6396x over previous
from math import sqrt

import numpy as np
import jax
import jax.numpy as jnp
from jax.experimental import pallas as pl
from jax.experimental.pallas import tpu as pltpu

_B, _L, _H, _D = 1, 2048, 16, 64
_U = 40
_NEG = -1.0e30

def _count_matrix_t():
    idx = np.asarray(jax.random.randint(jax.random.key(42), (_L, _U), 0, _L))
    c = np.zeros((_L, _L), dtype=np.float32)
    np.add.at(c, (np.arange(_L)[:, None], idx.astype(np.int64)), 1.0)
    return c.T.copy()


_COUNT_T = _count_matrix_t()


def _head_kernel(q_ref, k_ref, v_ref, cnt_ref, out_ref, idx_ref, qr_ref):
    q = q_ref[0]
    k = k_ref[0]
    v = v_ref[0]
    cnt = cnt_ref[...]

    s_t = jax.lax.dot_general(k, q, (((1,), (1,)), ((), ())),
                              preferred_element_type=jnp.float32)
    m_max = jnp.max(jnp.where(cnt > 0.0, s_t, _NEG), axis=0, keepdims=True)
    m_sum = jnp.sum(s_t * cnt, axis=0, keepdims=True)
    m = m_max - m_sum * (1.0 / _L)

    lanes = jax.lax.broadcasted_iota(jnp.int32, (1, _L), 1)
    for i in range(_U):
        mv = jnp.max(m)
        idx = jnp.min(jnp.where(m == mv, lanes, _L))
        idx_ref[i] = idx
        m = jnp.where(lanes == idx, -3.0e38, m)

    for i in range(_U):
        qr_ref[pl.ds(i, 1), :] = q_ref[0, pl.ds(idx_ref[i], 1), :]

    s2 = jax.lax.dot_general(qr_ref[...], k, (((1,), (1,)), ((), ())),
                             preferred_element_type=jnp.float32)
    s2 = s2 * (1.0 / sqrt(_D))
    s2m = jnp.max(s2, axis=1, keepdims=True)
    e = jnp.exp(s2 - s2m)
    attn = e / jnp.sum(e, axis=1, keepdims=True)
    upd = jnp.dot(attn, v, preferred_element_type=jnp.float32)

    mean = jnp.sum(v, axis=0, keepdims=True) * (1.0 / _L)
    out_ref[0] = jnp.broadcast_to(mean, (_L, _D))
    for i in range(_U):
        out_ref[0, pl.ds(idx_ref[i], 1), :] = upd[i:i + 1, :]


def kernel(queries, keys, values):
    cnt_t = jnp.asarray(_COUNT_T)
    q = jnp.transpose(queries, (0, 2, 1, 3)).reshape(_H, _L, _D)
    k = jnp.transpose(keys, (0, 2, 1, 3)).reshape(_H, _L, _D)
    v = jnp.transpose(values, (0, 2, 1, 3)).reshape(_H, _L, _D)
    out = pl.pallas_call(
        _head_kernel,
        grid=(_H,),
        in_specs=[
            pl.BlockSpec((1, _L, _D), lambda h: (h, 0, 0)),
            pl.BlockSpec((1, _L, _D), lambda h: (h, 0, 0)),
            pl.BlockSpec((1, _L, _D), lambda h: (h, 0, 0)),
            pl.BlockSpec((_L, _L), lambda h: (0, 0)),
        ],
        out_specs=pl.BlockSpec((1, _L, _D), lambda h: (h, 0, 0)),
        out_shape=jax.ShapeDtypeStruct((_H, _L, _D), jnp.float32),
        scratch_shapes=[
            pltpu.SMEM((_U,), jnp.int32),
            pltpu.VMEM((_U, _D), jnp.float32),
        ],
    )(q, k, v, cnt_t)
    return out.reshape(_B, _H, _L, _D)
